# baseline (speedup 1.0000x reference)
import functools

import jax
import jax.numpy as jnp
from jax import lax
from jax.experimental import pallas as pl
from jax.experimental.pallas import tpu as pltpu
from jax.experimental.pallas import tpu_sc as plsc

B = 16384
NF = 1_000_000
NW = 32
RANGE = 31256
NV = B // 16


def _sc_body(x1, idx1, w, part_hbm, idx_all, x_all, wr, tag, acc_v, wsem):
    cid = lax.axis_index("c")
    sid = lax.axis_index("s")
    wid = sid * 2 + cid
    base = wid * RANGE
    base_ld = jnp.minimum(base, NF - RANGE)
    shift = base - base_ld
    lane = lax.broadcasted_iota(jnp.int32, (16,), 0)

    pltpu.sync_copy(idx1, idx_all)
    pltpu.sync_copy(x1, x_all)
    wcopy = pltpu.async_copy(w.at[pl.ds(base_ld, RANGE)], wr, wsem)

    zero16 = jnp.zeros((16,), jnp.float32)
    one16 = jnp.ones((16,), jnp.float32)

    @plsc.parallel_loop(0, NV, unroll=8, carry=jnp.int32(0))
    def _scan(v, carry):
        iv = idx_all[pl.ds(v * 16, 16)]
        pv = v * 16 + lane
        li0 = iv - base
        m = (li0 >= 0) & (li0 < RANGE)
        li = jnp.where(m, li0, 0)
        plsc.store_scatter(tag, [li], pv, mask=m)
        return carry

    del _scan

    wcopy.wait()

    @plsc.parallel_loop(0, NV, unroll=4, carry=(zero16, zero16))
    def _win(v, carry):
        a, ns = carry
        iv = idx_all[pl.ds(v * 16, 16)]
        xv = x_all[pl.ds(v * 16, 16)]
        pv = v * 16 + lane
        li0 = iv - base
        m = (li0 >= 0) & (li0 < RANGE)
        li = jnp.where(m, li0, 0)
        t = plsc.load_gather(tag, [li], mask=m)
        win = m & (t == pv)
        wv = plsc.load_gather(wr, [li + shift], mask=win)
        a = a + jnp.where(win, wv * xv, zero16)
        ns = ns + jnp.where(m & (pv > t), one16, zero16)
        return a, ns

    acc, needsum = _win
    acc_v[...] = acc

    def _fix(nd, li, pv, depth):
        if depth == 0:
            return

        @pl.when(jnp.max(plsc.all_reduce_population_count(nd)) != 0)
        def _():
            plsc.store_scatter(tag, [li], pv, mask=nd)
            t = plsc.load_gather(tag, [li], mask=nd)
            _fix(nd & (pv > t), li, pv, depth - 1)

    @pl.when(jnp.max(needsum) != 0.0)
    def _():
        def fix_body(v, carry):
            iv = idx_all[pl.ds(v * 16, 16)]
            pv = v * 16 + lane
            li0 = iv - base
            m = (li0 >= 0) & (li0 < RANGE)
            li = jnp.where(m, li0, 0)
            t = plsc.load_gather(tag, [li], mask=m)
            _fix(m & (pv > t), li, pv, 15)
            return carry

        lax.fori_loop(0, NV, fix_body, jnp.int32(0))

        def repair_body(v, a):
            iv = idx_all[pl.ds(v * 16, 16)]
            xv = x_all[pl.ds(v * 16, 16)]
            pv = v * 16 + lane
            li0 = iv - base
            m = (li0 >= 0) & (li0 < RANGE)
            li = jnp.where(m, li0, 0)
            t2 = plsc.load_gather(tag, [li], mask=m)
            win = m & (t2 == pv)
            wv = plsc.load_gather(wr, [li + shift], mask=win)
            return a + jnp.where(win, wv * xv, zero16)

        acc_v[...] = lax.fori_loop(0, NV, repair_body, zero16)
    pltpu.sync_copy(acc_v, part_hbm.at[wid])


_sc_call = functools.partial(
    pl.kernel,
    out_type=jax.ShapeDtypeStruct((NW, 16), jnp.float32),
    mesh=plsc.VectorSubcoreMesh(core_axis_name="c", subcore_axis_name="s"),
    scratch_types=[
        pltpu.VMEM((B,), jnp.int32),
        pltpu.VMEM((B,), jnp.float32),
        pltpu.VMEM((RANGE,), jnp.float32),
        pltpu.VMEM((RANGE,), jnp.int32),
        pltpu.VMEM((16,), jnp.float32),
        pltpu.SemaphoreType.DMA,
    ],
    compiler_params=pltpu.CompilerParams(needs_layout_passes=False),
)(_sc_body)


def _tc_finish(p_ref, b_ref, o_ref):
    s = jnp.sum(p_ref[...]) + b_ref[0, 0]
    o_ref[...] = jnp.broadcast_to(s, (1, 1))


def kernel(x, idx, weight, bias, in_features):
    part = _sc_call(x, idx, weight)
    out = pl.pallas_call(
        _tc_finish,
        out_shape=jax.ShapeDtypeStruct((1, 1), jnp.float32),
    )(part, bias.reshape(1, 1).astype(jnp.float32))
    return out.reshape(1)


# speedup vs baseline: 1.0110x; 1.0110x over previous
import functools

import jax
import jax.numpy as jnp
from jax import lax
from jax.experimental import pallas as pl
from jax.experimental.pallas import tpu as pltpu
from jax.experimental.pallas import tpu_sc as plsc

B = 16384
NF = 1_000_000
NW = 32
RANGE = 31256
NV = B // 16


def _sc_body(x1, idx1, w, part_hbm, idx_all, x_all, wr, tag, acc_v, wsem,
             xsem):
    cid = lax.axis_index("c")
    sid = lax.axis_index("s")
    wid = sid * 2 + cid
    base = wid * RANGE
    base_ld = jnp.minimum(base, NF - RANGE)
    shift = base - base_ld
    lane = lax.broadcasted_iota(jnp.int32, (16,), 0)

    xcopy = pltpu.async_copy(x1, x_all, xsem)
    wcopy = pltpu.async_copy(w.at[pl.ds(base_ld, RANGE)], wr, wsem)
    pltpu.sync_copy(idx1, idx_all)

    zero16 = jnp.zeros((16,), jnp.float32)
    one16 = jnp.ones((16,), jnp.float32)

    @plsc.parallel_loop(0, NV, unroll=8, carry=jnp.int32(0))
    def _scan(v, carry):
        iv = idx_all[pl.ds(v * 16, 16)]
        pv = v * 16 + lane
        li0 = iv - base
        m = (li0 >= 0) & (li0 < RANGE)
        li = jnp.where(m, li0, 0)
        plsc.store_scatter(tag, [li], pv, mask=m)
        return carry

    del _scan

    wcopy.wait()
    xcopy.wait()

    @plsc.parallel_loop(0, NV, unroll=4, carry=(zero16, zero16))
    def _win(v, carry):
        a, ns = carry
        iv = idx_all[pl.ds(v * 16, 16)]
        xv = x_all[pl.ds(v * 16, 16)]
        pv = v * 16 + lane
        li0 = iv - base
        m = (li0 >= 0) & (li0 < RANGE)
        li = jnp.where(m, li0, 0)
        t = plsc.load_gather(tag, [li], mask=m)
        win = m & (t == pv)
        wv = plsc.load_gather(wr, [li + shift], mask=win)
        a = a + jnp.where(win, wv * xv, zero16)
        ns = ns + jnp.where(m & (pv > t), one16, zero16)
        return a, ns

    acc, needsum = _win
    acc_v[...] = acc

    def _fix(nd, li, pv, depth):
        if depth == 0:
            return

        @pl.when(jnp.max(plsc.all_reduce_population_count(nd)) != 0)
        def _():
            plsc.store_scatter(tag, [li], pv, mask=nd)
            t = plsc.load_gather(tag, [li], mask=nd)
            _fix(nd & (pv > t), li, pv, depth - 1)

    @pl.when(jnp.max(needsum) != 0.0)
    def _():
        def fix_body(v, carry):
            iv = idx_all[pl.ds(v * 16, 16)]
            pv = v * 16 + lane
            li0 = iv - base
            m = (li0 >= 0) & (li0 < RANGE)
            li = jnp.where(m, li0, 0)
            t = plsc.load_gather(tag, [li], mask=m)
            _fix(m & (pv > t), li, pv, 15)
            return carry

        lax.fori_loop(0, NV, fix_body, jnp.int32(0))

        def repair_body(v, a):
            iv = idx_all[pl.ds(v * 16, 16)]
            xv = x_all[pl.ds(v * 16, 16)]
            pv = v * 16 + lane
            li0 = iv - base
            m = (li0 >= 0) & (li0 < RANGE)
            li = jnp.where(m, li0, 0)
            t2 = plsc.load_gather(tag, [li], mask=m)
            win = m & (t2 == pv)
            wv = plsc.load_gather(wr, [li + shift], mask=win)
            return a + jnp.where(win, wv * xv, zero16)

        acc_v[...] = lax.fori_loop(0, NV, repair_body, zero16)
    pltpu.sync_copy(acc_v, part_hbm.at[wid])


_sc_call = functools.partial(
    pl.kernel,
    out_type=jax.ShapeDtypeStruct((NW, 16), jnp.float32),
    mesh=plsc.VectorSubcoreMesh(core_axis_name="c", subcore_axis_name="s"),
    scratch_types=[
        pltpu.VMEM((B,), jnp.int32),
        pltpu.VMEM((B,), jnp.float32),
        pltpu.VMEM((RANGE,), jnp.float32),
        pltpu.VMEM((RANGE,), jnp.int32),
        pltpu.VMEM((16,), jnp.float32),
        pltpu.SemaphoreType.DMA,
        pltpu.SemaphoreType.DMA,
    ],
    compiler_params=pltpu.CompilerParams(needs_layout_passes=False),
)(_sc_body)


def _tc_finish(p_ref, b_ref, o_ref):
    s = jnp.sum(p_ref[...]) + b_ref[0, 0]
    o_ref[...] = jnp.broadcast_to(s, (1, 1))


def kernel(x, idx, weight, bias, in_features):
    part = _sc_call(x, idx, weight)
    out = pl.pallas_call(
        _tc_finish,
        out_shape=jax.ShapeDtypeStruct((1, 1), jnp.float32),
    )(part, bias.reshape(1, 1).astype(jnp.float32))
    return out.reshape(1)
